# baseline (speedup 1.0000x reference)
import jax
import jax.numpy as jnp
from jax.experimental import pallas as pl
from jax.experimental.pallas import tpu as pltpu

_IN_DIMS = [4096, 2048, 1024, 512, 256, 128, 64, 32]
_NLAYERS = 7
_NUM_LEVELS = 4
_CB_SIZE = 256
_CB_DIM = 32
_MU = 0.25
_BATCH = 4096
_TILE = 256
_NT = _BATCH // _TILE

_PREC = jax.lax.Precision.DEFAULT
_EXACT = jax.lax.Precision.HIGHEST


def _mm(a, b, precision):
    return jax.lax.dot_general(a, b, (((1,), (0,)), ((), ())),
                               precision=precision,
                               preferred_element_type=jnp.float32)


def _enc_body(x_ref, *refs):
    enc_w = refs[0:_NLAYERS]
    enc_b = refs[_NLAYERS:2 * _NLAYERS]
    z_ref = refs[2 * _NLAYERS]

    h = x_ref[...]
    for i in range(_NLAYERS):
        h = _mm(h, enc_w[i][...], _PREC) + enc_b[i][...]
        if i < _NLAYERS - 1:
            h = jnp.maximum(h, 0.0)
    z_ref[...] = h


def _quant_body(z_ref, cb_ref, cbt_ref, zq_ref, idx_ref, qp_ref):
    z = z_ref[...]
    lane = jax.lax.broadcasted_iota(jnp.int32, (_BATCH, _CB_SIZE), 1)
    r = z
    quant = jnp.zeros_like(z)
    qp = jnp.float32(0.0)
    idx_cols = []
    for l in range(_NUM_LEVELS):
        cb = cb_ref[l]
        cbt = cbt_ref[l]
        rc = _mm(r, cbt, _PREC)
        d = (jnp.sum(r * r, axis=1, keepdims=True) - 2.0 * rc
             + jnp.sum(cb * cb, axis=1)[None, :])
        idx = jnp.argmin(d, axis=1).astype(jnp.int32)
        one_hot = (lane == idx[:, None]).astype(jnp.float32)
        q = _mm(one_hot, cb, _EXACT)
        qp = qp + jnp.sum((r - q) ** 2)
        quant = quant + q
        r = r - q
        idx_cols.append(idx)

    zq_ref[...] = z + (quant - z)
    idx_ref[...] = jnp.stack(idx_cols, axis=1)
    qp_ref[0, 0] = (1.0 + _MU) * qp


def _dec_body(zq_ref, x_ref, qp_ref, *refs):
    dec_w = refs[0:_NLAYERS]
    dec_b = refs[_NLAYERS:2 * _NLAYERS]
    xhat_ref, loss_ref, acc_ref = refs[2 * _NLAYERS:2 * _NLAYERS + 3]

    step = pl.program_id(0)
    h = zq_ref[...]
    for i in range(_NLAYERS):
        h = _mm(h, dec_w[i][...], _PREC) + dec_b[i][...]
        if i < _NLAYERS - 1:
            h = jnp.maximum(h, 0.0)
    xhat_ref[...] = h
    rp = jnp.sum((h - x_ref[...]) ** 2)

    @pl.when(step == 0)
    def _init():
        acc_ref[0] = rp

    @pl.when(step > 0)
    def _acc():
        acc_ref[0] = acc_ref[0] + rp

    @pl.when(step == _NT - 1)
    def _emit():
        loss_ref[0, 0] = (acc_ref[0] + qp_ref[0, 0]) / _BATCH


def _wspec(shape):
    nd = len(shape)
    return pl.BlockSpec(shape, lambda i, _nd=nd: (0,) * _nd)


def kernel(x, We0, We1, We2, We3, We4, We5, We6,
           be0, be1, be2, be3, be4, be5, be6,
           Wd0, Wd1, Wd2, Wd3, Wd4, Wd5, Wd6,
           bd0, bd1, bd2, bd3, bd4, bd5, bd6,
           codebooks):
    enc_w = [We0, We1, We2, We3, We4, We5, We6]
    enc_b = [b.reshape(1, -1) for b in (be0, be1, be2, be3, be4, be5, be6)]
    dec_w = [Wd0, Wd1, Wd2, Wd3, Wd4, Wd5, Wd6]
    dec_b = [b.reshape(1, -1) for b in (bd0, bd1, bd2, bd3, bd4, bd5, bd6)]
    cbt = jnp.transpose(codebooks, (0, 2, 1))

    cparams = pltpu.CompilerParams(
        dimension_semantics=("arbitrary",),
        vmem_limit_bytes=100 * 1024 * 1024,
    )

    z = pl.pallas_call(
        _enc_body,
        grid=(_NT,),
        in_specs=([pl.BlockSpec((_TILE, _IN_DIMS[0]), lambda i: (i, 0))]
                  + [_wspec(w.shape) for w in enc_w]
                  + [_wspec(b.shape) for b in enc_b]),
        out_specs=pl.BlockSpec((_TILE, _CB_DIM), lambda i: (i, 0)),
        out_shape=jax.ShapeDtypeStruct((_BATCH, _CB_DIM), jnp.float32),
        compiler_params=cparams,
    )(x, *enc_w, *enc_b)

    zq, idx, qp = pl.pallas_call(
        _quant_body,
        out_specs=[pl.BlockSpec(memory_space=pltpu.VMEM),
                   pl.BlockSpec(memory_space=pltpu.VMEM),
                   pl.BlockSpec(memory_space=pltpu.SMEM)],
        out_shape=[jax.ShapeDtypeStruct((_BATCH, _CB_DIM), jnp.float32),
                   jax.ShapeDtypeStruct((_BATCH, _NUM_LEVELS), jnp.int32),
                   jax.ShapeDtypeStruct((1, 1), jnp.float32)],
    )(z, codebooks, cbt)

    xhat, loss = pl.pallas_call(
        _dec_body,
        grid=(_NT,),
        in_specs=([pl.BlockSpec((_TILE, _CB_DIM), lambda i: (i, 0)),
                   pl.BlockSpec((_TILE, _IN_DIMS[0]), lambda i: (i, 0)),
                   pl.BlockSpec(memory_space=pltpu.SMEM)]
                  + [_wspec(w.shape) for w in dec_w]
                  + [_wspec(b.shape) for b in dec_b]),
        out_specs=[pl.BlockSpec((_TILE, _IN_DIMS[0]), lambda i: (i, 0)),
                   pl.BlockSpec((1, 1), lambda i: (0, 0),
                                memory_space=pltpu.SMEM)],
        out_shape=[jax.ShapeDtypeStruct((_BATCH, _IN_DIMS[0]), jnp.float32),
                   jax.ShapeDtypeStruct((1, 1), jnp.float32)],
        scratch_shapes=[pltpu.SMEM((1,), jnp.float32)],
        compiler_params=cparams,
    )(zq, x, qp, *dec_w, *dec_b)

    return xhat, loss[0, 0], idx


# speedup vs baseline: 1.0662x; 1.0662x over previous
import jax
import jax.numpy as jnp
from jax.experimental import pallas as pl
from jax.experimental.pallas import tpu as pltpu

_IN_DIMS = [4096, 2048, 1024, 512, 256, 128, 64, 32]
_NLAYERS = 7
_NUM_LEVELS = 4
_CB_SIZE = 256
_CB_DIM = 32
_MU = 0.25
_BATCH = 4096
_TILE = 256
_NT = _BATCH // _TILE
_TILE_E = 512
_NT_E = _BATCH // _TILE_E

_PREC = jax.lax.Precision.DEFAULT
_EXACT = jax.lax.Precision.HIGHEST


def _mm(a, b, precision):
    return jax.lax.dot_general(a, b, (((1,), (0,)), ((), ())),
                               precision=precision,
                               preferred_element_type=jnp.float32)


def _enc_body(x_ref, *refs):
    enc_w = refs[0:_NLAYERS]
    enc_b = refs[_NLAYERS:2 * _NLAYERS]
    z_ref = refs[2 * _NLAYERS]

    h = x_ref[...]
    for i in range(_NLAYERS):
        h = _mm(h, enc_w[i][...], _PREC) + enc_b[i][...]
        if i < _NLAYERS - 1:
            h = jnp.maximum(h, 0.0)
    z_ref[...] = h


def _quant_body(z_ref, cb_ref, cbt_ref, zq_ref, idx_ref, qp_ref):
    z = z_ref[...]
    lane = jax.lax.broadcasted_iota(jnp.int32, (_BATCH, _CB_SIZE), 1)
    r = z
    quant = jnp.zeros_like(z)
    qp = jnp.float32(0.0)
    idx_cols = []
    for l in range(_NUM_LEVELS):
        cb = cb_ref[l]
        cbt = cbt_ref[l]
        rc = _mm(r, cbt, _PREC)
        d = (jnp.sum(r * r, axis=1, keepdims=True) - 2.0 * rc
             + jnp.sum(cb * cb, axis=1)[None, :])
        dmin = jnp.min(d, axis=1, keepdims=True)
        idx = jnp.min(jnp.where(d == dmin, lane, _CB_SIZE), axis=1)
        one_hot = (lane == idx[:, None]).astype(jnp.float32)
        q = _mm(one_hot, cb, _EXACT)
        qp = qp + jnp.sum((r - q) ** 2)
        quant = quant + q
        r = r - q
        idx_cols.append(idx)

    zq_ref[...] = z + (quant - z)
    idx_ref[...] = jnp.stack(idx_cols, axis=1)
    qp_ref[0, 0] = (1.0 + _MU) * qp


def _dec_body(zq_ref, x_ref, qp_ref, *refs):
    dec_w = refs[0:_NLAYERS]
    dec_b = refs[_NLAYERS:2 * _NLAYERS]
    xhat_ref, loss_ref, acc_ref = refs[2 * _NLAYERS:2 * _NLAYERS + 3]

    step = pl.program_id(0)
    h = zq_ref[...]
    for i in range(_NLAYERS):
        h = _mm(h, dec_w[i][...], _PREC) + dec_b[i][...]
        if i < _NLAYERS - 1:
            h = jnp.maximum(h, 0.0)
    xhat_ref[...] = h
    rp = jnp.sum((h - x_ref[...]) ** 2)

    @pl.when(step == 0)
    def _init():
        acc_ref[0] = rp

    @pl.when(step > 0)
    def _acc():
        acc_ref[0] = acc_ref[0] + rp

    @pl.when(step == _NT - 1)
    def _emit():
        loss_ref[0, 0] = (acc_ref[0] + qp_ref[0, 0]) / _BATCH


def _wspec(shape):
    nd = len(shape)
    return pl.BlockSpec(shape, lambda i, _nd=nd: (0,) * _nd)


def kernel(x, We0, We1, We2, We3, We4, We5, We6,
           be0, be1, be2, be3, be4, be5, be6,
           Wd0, Wd1, Wd2, Wd3, Wd4, Wd5, Wd6,
           bd0, bd1, bd2, bd3, bd4, bd5, bd6,
           codebooks):
    enc_w = [We0, We1, We2, We3, We4, We5, We6]
    enc_b = [b.reshape(1, -1) for b in (be0, be1, be2, be3, be4, be5, be6)]
    dec_w = [Wd0, Wd1, Wd2, Wd3, Wd4, Wd5, Wd6]
    dec_b = [b.reshape(1, -1) for b in (bd0, bd1, bd2, bd3, bd4, bd5, bd6)]
    cbt = jnp.transpose(codebooks, (0, 2, 1))

    cparams = pltpu.CompilerParams(
        dimension_semantics=("arbitrary",),
        vmem_limit_bytes=100 * 1024 * 1024,
    )

    z = pl.pallas_call(
        _enc_body,
        grid=(_NT_E,),
        in_specs=([pl.BlockSpec((_TILE_E, _IN_DIMS[0]), lambda i: (i, 0))]
                  + [_wspec(w.shape) for w in enc_w]
                  + [_wspec(b.shape) for b in enc_b]),
        out_specs=pl.BlockSpec((_TILE_E, _CB_DIM), lambda i: (i, 0)),
        out_shape=jax.ShapeDtypeStruct((_BATCH, _CB_DIM), jnp.float32),
        compiler_params=cparams,
    )(x, *enc_w, *enc_b)

    zq, idx, qp = pl.pallas_call(
        _quant_body,
        out_specs=[pl.BlockSpec(memory_space=pltpu.VMEM),
                   pl.BlockSpec(memory_space=pltpu.VMEM),
                   pl.BlockSpec(memory_space=pltpu.SMEM)],
        out_shape=[jax.ShapeDtypeStruct((_BATCH, _CB_DIM), jnp.float32),
                   jax.ShapeDtypeStruct((_BATCH, _NUM_LEVELS), jnp.int32),
                   jax.ShapeDtypeStruct((1, 1), jnp.float32)],
    )(z, codebooks, cbt)

    xhat, loss = pl.pallas_call(
        _dec_body,
        grid=(_NT,),
        in_specs=([pl.BlockSpec((_TILE, _CB_DIM), lambda i: (i, 0)),
                   pl.BlockSpec((_TILE, _IN_DIMS[0]), lambda i: (i, 0)),
                   pl.BlockSpec(memory_space=pltpu.SMEM)]
                  + [_wspec(w.shape) for w in dec_w]
                  + [_wspec(b.shape) for b in dec_b]),
        out_specs=[pl.BlockSpec((_TILE, _IN_DIMS[0]), lambda i: (i, 0)),
                   pl.BlockSpec((1, 1), lambda i: (0, 0),
                                memory_space=pltpu.SMEM)],
        out_shape=[jax.ShapeDtypeStruct((_BATCH, _IN_DIMS[0]), jnp.float32),
                   jax.ShapeDtypeStruct((1, 1), jnp.float32)],
        scratch_shapes=[pltpu.SMEM((1,), jnp.float32)],
        compiler_params=cparams,
    )(zq, x, qp, *dec_w, *dec_b)

    return xhat, loss[0, 0], idx


# speedup vs baseline: 1.1353x; 1.0648x over previous
import jax
import jax.numpy as jnp
from jax.experimental import pallas as pl
from jax.experimental.pallas import tpu as pltpu

_IN_DIMS = [4096, 2048, 1024, 512, 256, 128, 64, 32]
_NLAYERS = 7
_NUM_LEVELS = 4
_CB_SIZE = 256
_CB_DIM = 32
_MU = 0.25
_BATCH = 4096
_TILE = 256
_NT = _BATCH // _TILE
_TILE_E = 512
_NT_E = _BATCH // _TILE_E

_PREC = jax.lax.Precision.DEFAULT
_EXACT = jax.lax.Precision.HIGHEST


def _mm(a, b, precision):
    return jax.lax.dot_general(a, b, (((1,), (0,)), ((), ())),
                               precision=precision,
                               preferred_element_type=jnp.float32)


def _enc_body(x_ref, *refs):
    enc_w = refs[0:_NLAYERS]
    enc_b = refs[_NLAYERS:2 * _NLAYERS]
    z_ref = refs[2 * _NLAYERS]

    h = x_ref[...]
    for i in range(_NLAYERS):
        h = _mm(h, enc_w[i][...], _PREC) + enc_b[i][...]
        if i < _NLAYERS - 1:
            h = jnp.maximum(h, 0.0)
    z_ref[...] = h


def _quant_body(z_ref, cb_ref, cbt2_ref, zq_ref, idx_ref, qp_ref):
    z = z_ref[...]
    lane = jax.lax.broadcasted_iota(jnp.int32, (_BATCH, _CB_SIZE), 1)
    r = z
    quant = jnp.zeros_like(z)
    qp = jnp.float32(0.0)
    idx_cols = []
    for l in range(_NUM_LEVELS):
        cb = cb_ref[l]
        rc2 = _mm(r, cbt2_ref[l], _PREC)
        d = (jnp.sum(r * r, axis=1, keepdims=True) - rc2
             + jnp.sum(cb * cb, axis=1)[None, :])
        dmin = jnp.min(d, axis=1, keepdims=True)
        idx = jnp.min(jnp.where(d == dmin, lane, _CB_SIZE), axis=1)
        one_hot = (lane == idx[:, None]).astype(jnp.float32)
        q = _mm(one_hot, cb, _PREC)
        qp = qp + jnp.sum((r - q) ** 2)
        quant = quant + q
        r = r - q
        idx_cols.append(idx)

    zq_ref[...] = z + (quant - z)
    idx_ref[...] = jnp.stack(idx_cols, axis=1)
    qp_ref[0, 0] = (1.0 + _MU) * qp


def _dec_body(zq_ref, x_ref, qp_ref, *refs):
    dec_w = refs[0:_NLAYERS]
    dec_b = refs[_NLAYERS:2 * _NLAYERS]
    xhat_ref, loss_ref, acc_ref = refs[2 * _NLAYERS:2 * _NLAYERS + 3]

    step = pl.program_id(0)
    h = zq_ref[...]
    for i in range(_NLAYERS):
        h = _mm(h, dec_w[i][...], _PREC) + dec_b[i][...]
        if i < _NLAYERS - 1:
            h = jnp.maximum(h, 0.0)
    xhat_ref[...] = h
    rp = jnp.sum((h - x_ref[...]) ** 2)

    @pl.when(step == 0)
    def _init():
        acc_ref[0] = rp

    @pl.when(step > 0)
    def _acc():
        acc_ref[0] = acc_ref[0] + rp

    @pl.when(step == _NT - 1)
    def _emit():
        loss_ref[0, 0] = (acc_ref[0] + qp_ref[0, 0]) / _BATCH


def _wspec(shape):
    nd = len(shape)
    return pl.BlockSpec(shape, lambda i, _nd=nd: (0,) * _nd)


def kernel(x, We0, We1, We2, We3, We4, We5, We6,
           be0, be1, be2, be3, be4, be5, be6,
           Wd0, Wd1, Wd2, Wd3, Wd4, Wd5, Wd6,
           bd0, bd1, bd2, bd3, bd4, bd5, bd6,
           codebooks):
    enc_w = [We0, We1, We2, We3, We4, We5, We6]
    enc_b = [b.reshape(1, -1) for b in (be0, be1, be2, be3, be4, be5, be6)]
    dec_w = [Wd0, Wd1, Wd2, Wd3, Wd4, Wd5, Wd6]
    dec_b = [b.reshape(1, -1) for b in (bd0, bd1, bd2, bd3, bd4, bd5, bd6)]
    cbt2 = 2.0 * jnp.transpose(codebooks, (0, 2, 1))

    cparams = pltpu.CompilerParams(
        dimension_semantics=("arbitrary",),
        vmem_limit_bytes=100 * 1024 * 1024,
    )

    z = pl.pallas_call(
        _enc_body,
        grid=(_NT_E,),
        in_specs=([pl.BlockSpec((_TILE_E, _IN_DIMS[0]), lambda i: (i, 0))]
                  + [_wspec(w.shape) for w in enc_w]
                  + [_wspec(b.shape) for b in enc_b]),
        out_specs=pl.BlockSpec((_TILE_E, _CB_DIM), lambda i: (i, 0)),
        out_shape=jax.ShapeDtypeStruct((_BATCH, _CB_DIM), jnp.float32),
        compiler_params=cparams,
    )(x, *enc_w, *enc_b)

    zq, idx, qp = pl.pallas_call(
        _quant_body,
        out_specs=[pl.BlockSpec(memory_space=pltpu.VMEM),
                   pl.BlockSpec(memory_space=pltpu.VMEM),
                   pl.BlockSpec(memory_space=pltpu.SMEM)],
        out_shape=[jax.ShapeDtypeStruct((_BATCH, _CB_DIM), jnp.float32),
                   jax.ShapeDtypeStruct((_BATCH, _NUM_LEVELS), jnp.int32),
                   jax.ShapeDtypeStruct((1, 1), jnp.float32)],
    )(z, codebooks, cbt2)

    xhat, loss = pl.pallas_call(
        _dec_body,
        grid=(_NT,),
        in_specs=([pl.BlockSpec((_TILE, _CB_DIM), lambda i: (i, 0)),
                   pl.BlockSpec((_TILE, _IN_DIMS[0]), lambda i: (i, 0)),
                   pl.BlockSpec(memory_space=pltpu.SMEM)]
                  + [_wspec(w.shape) for w in dec_w]
                  + [_wspec(b.shape) for b in dec_b]),
        out_specs=[pl.BlockSpec((_TILE, _IN_DIMS[0]), lambda i: (i, 0)),
                   pl.BlockSpec((1, 1), lambda i: (0, 0),
                                memory_space=pltpu.SMEM)],
        out_shape=[jax.ShapeDtypeStruct((_BATCH, _IN_DIMS[0]), jnp.float32),
                   jax.ShapeDtypeStruct((1, 1), jnp.float32)],
        scratch_shapes=[pltpu.SMEM((1,), jnp.float32)],
        compiler_params=cparams,
    )(zq, x, qp, *dec_w, *dec_b)

    return xhat, loss[0, 0], idx
